# Initial kernel scaffold; baseline (speedup 1.0000x reference)
import jax
import jax.numpy as jnp
from jax.experimental import pallas as pl


def kernel(input, gate_w, w_gate_proj, w_up_proj, w_down_proj):
    raise NotImplementedError("write your pallas kernel here")


# speedup vs baseline: 1.0627x; 1.0627x over previous
import functools

import jax
import jax.numpy as jnp
from jax.experimental import pallas as pl
from jax.experimental.pallas import tpu as pltpu

S = 4096
H = 2048
I = 1024
E = 16
K = 2
CAP = (2 * S * K) // E

BLK = 256


def _mlp_body(y_ref, wg_ref, wu_ref, wd_ref, out_ref):
    y = y_ref[0]
    dn = (((1,), (1,)), ((), ()))
    g = jax.lax.dot_general(y, wg_ref[0], dn, preferred_element_type=jnp.float32)
    u = jax.lax.dot_general(y, wu_ref[0], dn, preferred_element_type=jnp.float32)
    h = g * jax.nn.sigmoid(g) * u
    out_ref[0] = jax.lax.dot_general(h, wd_ref[0], dn,
                                     preferred_element_type=jnp.float32)


@functools.partial(jax.jit, static_argnames=("interpret",))
def _expert_mlp(y, w_gate, w_up, w_down, interpret=False):
    grid = (E, CAP // BLK)
    return pl.pallas_call(
        _mlp_body,
        grid=grid,
        in_specs=[
            pl.BlockSpec((1, BLK, H), lambda e, b: (e, b, 0)),
            pl.BlockSpec((1, I, H), lambda e, b: (e, 0, 0)),
            pl.BlockSpec((1, I, H), lambda e, b: (e, 0, 0)),
            pl.BlockSpec((1, H, I), lambda e, b: (e, 0, 0)),
        ],
        out_specs=pl.BlockSpec((1, BLK, H), lambda e, b: (e, b, 0)),
        out_shape=jax.ShapeDtypeStruct((E, CAP, H), jnp.float32),
        interpret=interpret,
    )(y, w_gate, w_up, w_down)


def kernel(input, gate_w, w_gate_proj, w_up_proj, w_down_proj):
    x = input
    gate_logits = x.astype(jnp.float32) @ gate_w.T
    gate_prob = jax.nn.softmax(gate_logits, axis=-1)
    topk_prob, topk_idx = jax.lax.top_k(gate_prob, K)
    flat_e = topk_idx.reshape(-1)
    order = jnp.argsort(flat_e)
    sorted_e = flat_e[order]
    first = jnp.searchsorted(sorted_e, sorted_e, side='left')
    slots_sorted = (jnp.arange(S * K) - first).astype(jnp.int32)
    slot = jnp.zeros((S * K,), jnp.int32).at[order].set(slots_sorted)
    e32 = flat_e.astype(jnp.int32)
    keep = slot < CAP
    pos = e32 * CAP + slot
    pos_safe = jnp.where(keep, pos, E * CAP)
    x_rep = jnp.repeat(x, K, axis=0)
    y = jnp.zeros((E * CAP + 1, H), x.dtype).at[pos_safe].set(x_rep)[:-1]

    yd = y.reshape(E, CAP, H)
    expert_out = _expert_mlp(yd, w_gate_proj, w_up_proj, w_down_proj)

    combine_weights = jnp.where(keep.reshape(S, K), topk_prob, 0.0)
    scatter_index = jnp.where(keep, pos, -1).reshape(S, K).T
    out_flat = expert_out.reshape(E * CAP, H)
    idx = jnp.clip(scatter_index, 0)
    gathered = out_flat[idx]
    combined = jnp.sum(gathered * combine_weights.T[:, :, None], axis=0)
    router_loss = jnp.zeros((1,), jnp.float32)
    return combined, combine_weights, router_loss, gate_logits


# speedup vs baseline: 1.0636x; 1.0009x over previous
import functools

import jax
import jax.numpy as jnp
from jax.experimental import pallas as pl
from jax.experimental.pallas import tpu as pltpu

S = 4096
H = 2048
I = 1024
E = 16
K = 2
CAP = (2 * S * K) // E

BLK = 256


def _mlp_body(y_ref, wg_ref, wu_ref, wd_ref, out_ref):
    y = y_ref[0]
    dn = (((1,), (1,)), ((), ()))
    g = jax.lax.dot_general(y, wg_ref[0], dn, preferred_element_type=jnp.float32)
    u = jax.lax.dot_general(y, wu_ref[0], dn, preferred_element_type=jnp.float32)
    h = (g * jax.nn.sigmoid(g) * u).astype(jnp.bfloat16)
    out_ref[0] = jax.lax.dot_general(h, wd_ref[0], dn,
                                     preferred_element_type=jnp.float32)


@functools.partial(jax.jit, static_argnames=("interpret",))
def _expert_mlp(y, w_gate, w_up, w_down, interpret=False):
    grid = (E, CAP // BLK)
    return pl.pallas_call(
        _mlp_body,
        grid=grid,
        in_specs=[
            pl.BlockSpec((1, BLK, H), lambda e, b: (e, b, 0)),
            pl.BlockSpec((1, I, H), lambda e, b: (e, 0, 0)),
            pl.BlockSpec((1, I, H), lambda e, b: (e, 0, 0)),
            pl.BlockSpec((1, H, I), lambda e, b: (e, 0, 0)),
        ],
        out_specs=pl.BlockSpec((1, BLK, H), lambda e, b: (e, b, 0)),
        out_shape=jax.ShapeDtypeStruct((E, CAP, H), jnp.float32),
        interpret=interpret,
    )(y, w_gate, w_up, w_down)


def kernel(input, gate_w, w_gate_proj, w_up_proj, w_down_proj):
    x = input
    gate_logits = x.astype(jnp.float32) @ gate_w.T
    gate_prob = jax.nn.softmax(gate_logits, axis=-1)
    topk_prob, topk_idx = jax.lax.top_k(gate_prob, K)
    flat_e = topk_idx.reshape(-1)
    order = jnp.argsort(flat_e)
    sorted_e = flat_e[order]
    first = jnp.searchsorted(sorted_e, sorted_e, side='left')
    slots_sorted = (jnp.arange(S * K) - first).astype(jnp.int32)
    slot = jnp.zeros((S * K,), jnp.int32).at[order].set(slots_sorted)
    e32 = flat_e.astype(jnp.int32)
    keep = slot < CAP
    pos = e32 * CAP + slot
    pos_safe = jnp.where(keep, pos, E * CAP)
    x_rep = jnp.repeat(x, K, axis=0)
    y = jnp.zeros((E * CAP + 1, H), x.dtype).at[pos_safe].set(x_rep)[:-1]

    yd = y.reshape(E, CAP, H)
    expert_out = _expert_mlp(yd, w_gate_proj, w_up_proj, w_down_proj)

    combine_weights = jnp.where(keep.reshape(S, K), topk_prob, 0.0)
    scatter_index = jnp.where(keep, pos, -1).reshape(S, K).T
    out_flat = expert_out.reshape(E * CAP, H)
    idx = jnp.clip(scatter_index, 0)
    gathered = out_flat[idx]
    combined = jnp.sum(gathered * combine_weights.T[:, :, None], axis=0)
    router_loss = jnp.zeros((1,), jnp.float32)
    return combined, combine_weights, router_loss, gate_logits
